# baseline (speedup 1.0000x reference)
import functools

import jax
import jax.numpy as jnp
from jax.experimental import pallas as pl
from jax.experimental.pallas import tpu as pltpu

G, IN_DIM, OUT_DIM, COLS = 4, 4096, 4096, 256
BK = 1024
NK = IN_DIM // BK
HALF = OUT_DIM // 2


def _body(x_ref, w0_ref, w1_ref, b_ref, o_ref):
    k = pl.program_id(1)
    x = x_ref[0]
    dn = (((0,), (0,)), ((), ()))
    acc0 = jax.lax.dot_general(w0_ref[0], x, dimension_numbers=dn,
                               preferred_element_type=jnp.float32)
    acc1 = jax.lax.dot_general(w1_ref[0], x, dimension_numbers=dn,
                               preferred_element_type=jnp.float32)

    @pl.when(k == 0)
    def _():
        o_ref[0, :HALF] = acc0 + b_ref[0, :HALF]
        o_ref[0, HALF:] = acc1 + b_ref[0, HALF:]

    @pl.when(k > 0)
    def _():
        o_ref[0, :HALF] += acc0
        o_ref[0, HALF:] += acc1


@functools.partial(jax.jit, static_argnames=("interpret",))
def kernel(inputs, W, b, interpret=False):
    return pl.pallas_call(
        _body,
        grid=(G, NK),
        in_specs=[
            pl.BlockSpec((1, BK, COLS), lambda g, k: (g, k, 0)),
            pl.BlockSpec((1, BK, HALF), lambda g, k: (g, k, 0)),
            pl.BlockSpec((1, BK, HALF), lambda g, k: (g, k, 1)),
            pl.BlockSpec((1, OUT_DIM, 1), lambda g, k: (g, 0, 0)),
        ],
        out_specs=pl.BlockSpec((1, OUT_DIM, COLS), lambda g, k: (g, 0, 0)),
        out_shape=jax.ShapeDtypeStruct((G, OUT_DIM, COLS), jnp.float32),
        compiler_params=pltpu.CompilerParams(
            dimension_semantics=("parallel", "arbitrary"),
            fuse_transposed_lhs_in_matmul=True,
        ),
        interpret=interpret,
    )(inputs, W, W, b)


# speedup vs baseline: 1.1457x; 1.1457x over previous
import functools

import jax
import jax.numpy as jnp
from jax.experimental import pallas as pl
from jax.experimental.pallas import tpu as pltpu

G, IN_DIM, OUT_DIM, COLS = 4, 4096, 4096, 256
BK = 1024
NK = IN_DIM // BK
HALF = OUT_DIM // 2


def _body(x_ref, w0_ref, w1_ref, b_ref, o_ref):
    k = pl.program_id(1)
    x = x_ref[0]
    dn = (((0,), (0,)), ((), ()))
    acc0 = jax.lax.dot_general(w0_ref[0], x, dimension_numbers=dn,
                               preferred_element_type=jnp.float32)
    acc1 = jax.lax.dot_general(w1_ref[0], x, dimension_numbers=dn,
                               preferred_element_type=jnp.float32)

    @pl.when(k == 0)
    def _():
        bias = b_ref[0, 0].reshape(OUT_DIM, 1)
        o_ref[0, :HALF] = acc0 + bias[:HALF]
        o_ref[0, HALF:] = acc1 + bias[HALF:]

    @pl.when(k > 0)
    def _():
        o_ref[0, :HALF] += acc0
        o_ref[0, HALF:] += acc1


@functools.partial(jax.jit, static_argnames=("interpret",))
def kernel(inputs, W, b, interpret=False):
    return pl.pallas_call(
        _body,
        grid=(G, NK),
        in_specs=[
            pl.BlockSpec((1, BK, COLS), lambda g, k: (g, k, 0)),
            pl.BlockSpec((1, BK, HALF), lambda g, k: (g, k, 0)),
            pl.BlockSpec((1, BK, HALF), lambda g, k: (g, k, 1)),
            pl.BlockSpec((1, 1, OUT_DIM), lambda g, k: (g, 0, 0)),
        ],
        out_specs=pl.BlockSpec((1, OUT_DIM, COLS), lambda g, k: (g, 0, 0)),
        out_shape=jax.ShapeDtypeStruct((G, OUT_DIM, COLS), jnp.float32),
        compiler_params=pltpu.CompilerParams(
            dimension_semantics=("parallel", "arbitrary"),
            vmem_limit_bytes=100 * 1024 * 1024,
        ),
        interpret=interpret,
    )(inputs, W, W, b.reshape(G, 1, OUT_DIM))
